# Initial kernel scaffold; baseline (speedup 1.0000x reference)
import jax
import jax.numpy as jnp
from jax.experimental import pallas as pl

N = 4096
D = 512
K = 32
BLOCK_ROWS = 128

NEG = jnp.float32(-3.0e38)


def _normalize_body(x_ref, out_ref):
    x = x_ref[...]
    norm = jnp.sqrt(jnp.sum(x * x, axis=1, keepdims=True))
    out_ref[...] = x / jnp.maximum(norm, 1e-12)


def _knn_body(xb_ref, xall_ref, out_ref):
    a = xb_ref[...]
    b = xall_ref[...]
    s = jax.lax.dot_general(
        a, b, (((1,), (1,)), ((), ())), preferred_element_type=jnp.float32
    )
    m = jnp.max(s, axis=1, keepdims=True)
    for _ in range(K - 1):
        m = jnp.max(jnp.where(s < m, s, NEG), axis=1, keepdims=True)
    out_ref[...] = jnp.where(s >= m, s, 0.0)


@jax.jit
def kernel(x):
    xn = pl.pallas_call(
        _normalize_body,
        out_shape=jax.ShapeDtypeStruct((N, D), jnp.float32),
        grid=(8,),
        in_specs=[pl.BlockSpec((N // 8, D), lambda i: (i, 0))],
        out_specs=pl.BlockSpec((N // 8, D), lambda i: (i, 0)),
    )(x)
    out = pl.pallas_call(
        _knn_body,
        out_shape=jax.ShapeDtypeStruct((N, N), jnp.float32),
        grid=(N // BLOCK_ROWS,),
        in_specs=[
            pl.BlockSpec((BLOCK_ROWS, D), lambda i: (i, 0)),
            pl.BlockSpec((N, D), lambda i: (0, 0)),
        ],
        out_specs=pl.BlockSpec((BLOCK_ROWS, N), lambda i: (i, 0)),
    )(xn, xn)
    return out


# speedup vs baseline: 24.1321x; 24.1321x over previous
import jax
import jax.numpy as jnp
from jax.experimental import pallas as pl

N = 4096
D = 512
K = 32
BLOCK_ROWS = 128

NEG = -3.0e38


def _normalize_body(x_ref, out_ref):
    x = x_ref[...]
    norm = jnp.sqrt(jnp.sum(x * x, axis=1, keepdims=True))
    out_ref[...] = x / jnp.maximum(norm, 1e-12)


def _knn_body(xb_ref, xall_ref, out_ref):
    a = xb_ref[...]
    b = xall_ref[...]
    s = jax.lax.dot_general(
        a, b, (((1,), (1,)), ((), ())), preferred_element_type=jnp.float32
    )
    m = jnp.max(s, axis=1, keepdims=True)
    for _ in range(K - 1):
        m = jnp.max(jnp.where(s < m, s, NEG), axis=1, keepdims=True)
    out_ref[...] = jnp.where(s >= m, s, 0.0)


@jax.jit
def kernel(x):
    xn = pl.pallas_call(
        _normalize_body,
        out_shape=jax.ShapeDtypeStruct((N, D), jnp.float32),
        grid=(8,),
        in_specs=[pl.BlockSpec((N // 8, D), lambda i: (i, 0))],
        out_specs=pl.BlockSpec((N // 8, D), lambda i: (i, 0)),
    )(x)
    out = pl.pallas_call(
        _knn_body,
        out_shape=jax.ShapeDtypeStruct((N, N), jnp.float32),
        grid=(N // BLOCK_ROWS,),
        in_specs=[
            pl.BlockSpec((BLOCK_ROWS, D), lambda i: (i, 0)),
            pl.BlockSpec((N, D), lambda i: (0, 0)),
        ],
        out_specs=pl.BlockSpec((BLOCK_ROWS, N), lambda i: (i, 0)),
    )(xn, xn)
    return out


# speedup vs baseline: 28.4148x; 1.1775x over previous
import jax
import jax.numpy as jnp
from jax.experimental import pallas as pl

N = 4096
D = 512
K = 32
BLOCK_ROWS = 128

NEG = -3.0e38


def _normalize_body(x_ref, out_ref):
    x = x_ref[...]
    norm = jnp.sqrt(jnp.sum(x * x, axis=1, keepdims=True))
    out_ref[...] = x / jnp.maximum(norm, 1e-12)


BISECT_STEPS = 20
FINISH_STEPS = 4


def _knn_body(xb_ref, xall_ref, out_ref):
    a = xb_ref[...]
    b = xall_ref[...]
    s = jax.lax.dot_general(
        a, b, (((1,), (1,)), ((), ())), preferred_element_type=jnp.float32
    )
    rows = s.shape[0]
    lo = jnp.full((rows, 1), -1.05, jnp.float32)
    hi = jnp.full((rows, 1), 1.05, jnp.float32)
    c_hi = jnp.zeros((rows, 1), jnp.float32)
    kf = jnp.float32(K)
    for _ in range(BISECT_STEPS):
        mid = 0.5 * (lo + hi)
        cnt = jnp.sum(jnp.where(s >= mid, 1.0, 0.0), axis=1, keepdims=True)
        pred = cnt >= kf
        lo = jnp.where(pred, mid, lo)
        c_hi = jnp.where(pred, c_hi, cnt)
        hi = jnp.where(pred, hi, mid)
    m = hi
    c = c_hi
    for _ in range(FINISH_STEPS):
        take = c < kf
        nm = jnp.max(jnp.where(s < m, s, NEG), axis=1, keepdims=True)
        m = jnp.where(take, nm, m)
        c = c + jnp.where(take, 1.0, 0.0)
    out_ref[...] = jnp.where(s >= m, s, 0.0)


@jax.jit
def kernel(x):
    xn = pl.pallas_call(
        _normalize_body,
        out_shape=jax.ShapeDtypeStruct((N, D), jnp.float32),
        grid=(8,),
        in_specs=[pl.BlockSpec((N // 8, D), lambda i: (i, 0))],
        out_specs=pl.BlockSpec((N // 8, D), lambda i: (i, 0)),
    )(x)
    out = pl.pallas_call(
        _knn_body,
        out_shape=jax.ShapeDtypeStruct((N, N), jnp.float32),
        grid=(N // BLOCK_ROWS,),
        in_specs=[
            pl.BlockSpec((BLOCK_ROWS, D), lambda i: (i, 0)),
            pl.BlockSpec((N, D), lambda i: (0, 0)),
        ],
        out_specs=pl.BlockSpec((BLOCK_ROWS, N), lambda i: (i, 0)),
    )(xn, xn)
    return out


# speedup vs baseline: 32.2335x; 1.1344x over previous
import jax
import jax.numpy as jnp
from jax.experimental import pallas as pl

N = 4096
D = 512
K = 32
BLOCK_ROWS = 128

NEG = -3.0e38


def _normalize_body(x_ref, out_ref):
    x = x_ref[...]
    norm = jnp.sqrt(jnp.sum(x * x, axis=1, keepdims=True))
    out_ref[...] = x / jnp.maximum(norm, 1e-12)


BISECT_STEPS = 14
FINISH_STEPS = 6


def _knn_body(xb_ref, xall_ref, out_ref):
    a = xb_ref[...]
    b = xall_ref[...]
    s = jax.lax.dot_general(
        a, b, (((1,), (1,)), ((), ())), preferred_element_type=jnp.float32
    )
    rows = s.shape[0]
    lo = jnp.full((rows, 1), -1.05, jnp.float32)
    hi = jnp.full((rows, 1), 1.05, jnp.float32)
    c_hi = jnp.zeros((rows, 1), jnp.float32)
    kf = jnp.float32(K)
    for _ in range(BISECT_STEPS):
        mid = 0.5 * (lo + hi)
        cnt = jnp.sum(jnp.where(s >= mid, 1.0, 0.0), axis=1, keepdims=True)
        pred = cnt >= kf
        lo = jnp.where(pred, mid, lo)
        c_hi = jnp.where(pred, c_hi, cnt)
        hi = jnp.where(pred, hi, mid)
    m = hi
    c = c_hi
    for _ in range(FINISH_STEPS):
        take = c < kf
        nm = jnp.max(jnp.where(s < m, s, NEG), axis=1, keepdims=True)
        m = jnp.where(take, nm, m)
        c = c + jnp.where(take, 1.0, 0.0)
    out_ref[...] = jnp.where(s >= m, s, 0.0)


@jax.jit
def kernel(x):
    xn = pl.pallas_call(
        _normalize_body,
        out_shape=jax.ShapeDtypeStruct((N, D), jnp.float32),
        grid=(8,),
        in_specs=[pl.BlockSpec((N // 8, D), lambda i: (i, 0))],
        out_specs=pl.BlockSpec((N // 8, D), lambda i: (i, 0)),
    )(x)
    out = pl.pallas_call(
        _knn_body,
        out_shape=jax.ShapeDtypeStruct((N, N), jnp.float32),
        grid=(N // BLOCK_ROWS,),
        in_specs=[
            pl.BlockSpec((BLOCK_ROWS, D), lambda i: (i, 0)),
            pl.BlockSpec((N, D), lambda i: (0, 0)),
        ],
        out_specs=pl.BlockSpec((BLOCK_ROWS, N), lambda i: (i, 0)),
    )(xn, xn)
    return out
